# baseline (speedup 1.0000x reference)
import functools

import jax
import jax.numpy as jnp
from jax import lax
from jax.experimental import pallas as pl
from jax.experimental.pallas import tpu as pltpu
from jax.experimental.pallas import tpu_sc as plsc

B, S = 4096, 200
D = 64
N = B * S
NW = 32
ROWS_PER_W = N // NW
CHUNK = 128
NCHUNK = ROWS_PER_W // CHUNK


def _make_kernel():
    mesh = plsc.VectorSubcoreMesh(core_axis_name="c", subcore_axis_name="s")

    @functools.partial(
        pl.kernel,
        out_type=jax.ShapeDtypeStruct((N, D), jnp.float32),
        mesh=mesh,
        scratch_types=[
            pltpu.VMEM((NCHUNK, CHUNK), jnp.int32),
            pltpu.VMEM((CHUNK, D), jnp.float32),
            pltpu.SemaphoreType.DMA,
        ],
        compiler_params=pltpu.CompilerParams(use_tc_tiling_on_sc=False),
    )
    def emb(tid_hbm, table_hbm, out_hbm, idx_v, rows_v, gsem):
        wid = lax.axis_index("s") * 2 + lax.axis_index("c")
        pltpu.sync_copy(tid_hbm.at[pl.ds(wid * NCHUNK, NCHUNK)], idx_v)
        row_base = wid * ROWS_PER_W

        def body(c, _):
            pltpu.async_copy(table_hbm.at[idx_v.at[c]], rows_v, gsem).wait()
            pltpu.sync_copy(
                rows_v, out_hbm.at[pl.ds(row_base + c * CHUNK, CHUNK)]
            )
            return ()

        lax.fori_loop(0, NCHUNK, body, ())

    return emb


_emb = _make_kernel()


@jax.jit
def kernel(token_ids, weight):
    tid = token_ids.reshape(NW * NCHUNK, CHUNK)
    out = _emb(tid, weight)
    return out.reshape(B, S, D)


# speedup vs baseline: 1.1146x; 1.1146x over previous
import functools

import jax
import jax.numpy as jnp
from jax import lax
from jax.experimental import pallas as pl
from jax.experimental.pallas import tpu as pltpu
from jax.experimental.pallas import tpu_sc as plsc

B, S = 4096, 200
D = 64
N = B * S
NW = 32
ROWS_PER_W = N // NW
CHUNK = 128
NCHUNK = ROWS_PER_W // CHUNK
G = 8
NGRP = NCHUNK // G


def _make_kernel():
    mesh = plsc.VectorSubcoreMesh(core_axis_name="c", subcore_axis_name="s")

    @functools.partial(
        pl.kernel,
        out_type=jax.ShapeDtypeStruct((N, D), jnp.float32),
        mesh=mesh,
        scratch_types=[
            pltpu.VMEM((NCHUNK, CHUNK), jnp.int32),
            pltpu.VMEM((G, CHUNK, D), jnp.float32),
            pltpu.SemaphoreType.DMA((G,)),
            pltpu.SemaphoreType.DMA,
        ],
        compiler_params=pltpu.CompilerParams(use_tc_tiling_on_sc=False),
    )
    def emb(tid_hbm, table_hbm, out_hbm, idx_v, rows_v, gsem, wsem):
        wid = lax.axis_index("s") * 2 + lax.axis_index("c")
        pltpu.sync_copy(tid_hbm.at[pl.ds(wid * NCHUNK, NCHUNK)], idx_v)
        row_base = wid * ROWS_PER_W

        def body(grp, _):
            c0 = grp * G
            gathers = [
                pltpu.async_copy(
                    table_hbm.at[idx_v.at[c0 + b]], rows_v.at[b], gsem.at[b]
                )
                for b in range(G)
            ]
            wbs = []
            for b in range(G):
                gathers[b].wait()
                wbs.append(
                    pltpu.async_copy(
                        rows_v.at[b],
                        out_hbm.at[pl.ds(row_base + (c0 + b) * CHUNK, CHUNK)],
                        wsem,
                    )
                )
            for wb in wbs:
                wb.wait()
            return ()

        lax.fori_loop(0, NGRP, body, ())

    return emb


_emb = _make_kernel()


@jax.jit
def kernel(token_ids, weight):
    tid = token_ids.reshape(NW * NCHUNK, CHUNK)
    out = _emb(tid, weight)
    return out.reshape(B, S, D)
